# baseline (speedup 1.0000x reference)
import functools

import numpy as np
import jax
import jax.numpy as jnp
from jax import lax
from jax.experimental import pallas as pl
from jax.experimental.pallas import tpu as pltpu
from jax.experimental.pallas import tpu_sc as plsc

_H = 16
_DH = 128
_E = 16
_K = 8
_SIGMA = 9.0
_MARGIN = 1.0 / (_E * 2)
_BF = jnp.bfloat16
_F32 = jnp.float32


def _mxu(a16, b16):
    return lax.dot_general(a16, b16, (((1,), (1,)), ((), ())),
                           preferred_element_type=_F32)


def _q_body(qst_ref, wq_ref, bq_ref, q_ref):
    q_ref[...] = _mxu(qst_ref[...], wq_ref[...]) + bq_ref[0]


def _attn_body(data_ref, wk_ref, wv_ref, qh_ref, bk_ref, bv_ref,
               ctx_ref, s_ref):
    T = data_ref.shape[1]
    NT = T // 512
    scale = np.float32(1.0 / np.sqrt(_DH))
    for i in range(NT):
        dt = data_ref[0, i * 512:(i + 1) * 512, :]
        for jn in range(4):
            kt = _mxu(dt, wk_ref[jn * 512:(jn + 1) * 512, :])
            kt = kt + bk_ref[0, jn * 512:(jn + 1) * 512]
            k16f = kt.astype(_BF).astype(_F32)
            for hh in range(4):
                h = jn * 4 + hh
                qrow = qh_ref[0, 0, h * _DH:(h + 1) * _DH].astype(_F32)
                prod = k16f[:, hh * _DH:(hh + 1) * _DH] * qrow[None, :]
                s_ref[i * 512:(i + 1) * 512, h:h + 1] = (
                    jnp.sum(prod, axis=1, keepdims=True) * scale)
    s = s_ref[...]
    mx = jnp.max(s, axis=0, keepdims=True)
    e = jnp.exp(s - mx)
    a = e / jnp.sum(e, axis=0, keepdims=True)
    a16f = a.astype(_BF).astype(_F32)
    parts = [jnp.zeros((1, _DH), _F32) for _ in range(_H)]
    for i in range(NT):
        dt = data_ref[0, i * 512:(i + 1) * 512, :]
        at = lax.slice(a16f, (i * 512, 0), ((i + 1) * 512, _H))
        for jn in range(4):
            vt = _mxu(dt, wv_ref[jn * 512:(jn + 1) * 512, :])
            vt = vt + bv_ref[0, jn * 512:(jn + 1) * 512]
            v16f = vt.astype(_BF).astype(_F32)
            for hh in range(4):
                h = jn * 4 + hh
                prod = v16f[:, hh * _DH:(hh + 1) * _DH] * at[:, h:h + 1]
                parts[h] = parts[h] + jnp.sum(prod, axis=0, keepdims=True)
    for h in range(_H):
        ctx_ref[0, :, h * _DH:(h + 1) * _DH] = parts[h]


def _head_body(T_grid, ctx16_ref, outw_ref, outb_ref, rw_ref, rb_ref,
               gcw_ref, gcb_ref, gww_ref, gwb_ref, base_ref,
               logits_ref, c_ref, iw_ref, pk_ref, tw_ref):
    ctx16 = ctx16_ref[...]
    C = outw_ref.shape[0]
    for jn in range(C // 512):
        tw_ref[:, jn * 512:(jn + 1) * 512] = _mxu(
            ctx16, outw_ref[jn * 512:(jn + 1) * 512, :])
    tw16f = (tw_ref[...] + outb_ref[0]).astype(_BF).astype(_F32)
    rwf = rw_ref[...].astype(_F32)
    gcwf = gcw_ref[...].astype(_F32)
    gwwf = gww_ref[...].astype(_F32)
    inv_t = np.float32(1.0 / (T_grid - 1.0))
    for e_ in range(_E):
        lg = jnp.sum(tw16f * rwf[e_:e_ + 1, :], axis=1, keepdims=True)
        logits_ref[:, e_:e_ + 1] = lg + rb_ref[0, e_]
        gc = jnp.sum(tw16f * gcwf[e_:e_ + 1, :], axis=1, keepdims=True)
        cpred = jnp.tanh(gc + gcb_ref[0, e_]) * np.float32(_MARGIN)
        c = jnp.clip(base_ref[0, e_] + cpred, 0.0, 1.0)
        gw = jnp.sum(tw16f * gwwf[e_:e_ + 1, :], axis=1, keepdims=True)
        width = jax.nn.sigmoid(gw + gwb_ref[0, e_])
        w = jnp.maximum(width, 0.09) / np.float32(_SIGMA)
        iw = 1.0 / (2.0 * w * w)
        gstar = jnp.floor(c * np.float32(T_grid - 1.0) + 0.5) * inv_t
        dstar = gstar - c
        c_ref[:, e_:e_ + 1] = c
        iw_ref[:, e_:e_ + 1] = iw
        pk_ref[:, e_:e_ + 1] = dstar * dstar * iw


def _make_sc_weight(B, T):
    nchunk = T // 16
    inv = np.float32(1.0 / (T - 1))
    mesh = plsc.VectorSubcoreMesh(core_axis_name="c", subcore_axis_name="s")

    @functools.partial(
        pl.kernel, mesh=mesh,
        compiler_params=pltpu.CompilerParams(needs_layout_passes=False),
        out_type=jax.ShapeDtypeStruct((B, _K, T), jnp.float32),
        scratch_types=[
            pltpu.VMEM((16,), jnp.float32),
            pltpu.VMEM((16,), jnp.float32),
            pltpu.VMEM((16,), jnp.float32),
            pltpu.VMEM((16,), jnp.float32),
            pltpu.VMEM((T,), jnp.float32),
        ],
    )
    def k(logits_hbm, c_hbm, iw_hbm, pk_hbm, out_hbm,
          lrow, crow, iwrow, pkrow, buf):
        cid = lax.axis_index("c")
        sid = lax.axis_index("s")
        wid = sid * 2 + cid
        b = wid // 2
        half = wid % 2
        pltpu.sync_copy(logits_hbm.at[b], lrow)
        pltpu.sync_copy(c_hbm.at[b], crow)
        pltpu.sync_copy(iw_hbm.at[b], iwrow)
        pltpu.sync_copy(pk_hbm.at[b], pkrow)
        l = lrow[...]
        mx = jnp.max(l)
        e = jnp.exp(l - mx)
        p = e / jnp.sum(e)
        lanes = lax.iota(jnp.int32, 16)
        idx_vec = jnp.zeros((16,), jnp.int32)
        pw = p
        for kk in range(_K):
            mk = jnp.max(pw)
            ii = plsc.all_reduce_ffs(pw == mk)
            idx_vec = jnp.where(lanes == kk, ii, idx_vec)
            pw = jnp.where(lanes == ii, np.float32(-3.0e38), pw)
        csel = plsc.load_gather(crow, [idx_vec])
        iwsel = plsc.load_gather(iwrow, [idx_vec])
        pksel = plsc.load_gather(pkrow, [idx_vec])
        for j in range(_K // 2):
            slot = half * (_K // 2) + j
            smask = lanes == slot
            ck = jnp.sum(jnp.where(smask, csel, np.float32(0.0)))
            iwk = jnp.sum(jnp.where(smask, iwsel, np.float32(0.0)))
            pkk = jnp.sum(jnp.where(smask, pksel, np.float32(0.0)))

            def body(i, carry):
                g = (lax.iota(jnp.int32, 16) + i * 16).astype(jnp.float32) * inv
                dlt = g - ck
                buf[pl.ds(i * 16, 16)] = jnp.exp(pkk - dlt * dlt * iwk)
                return carry

            lax.fori_loop(0, nchunk, body, 0)
            pltpu.sync_copy(buf, out_hbm.at[b, slot])

    return k


def kernel(qst, data, in_proj_w, in_proj_b, out_w, out_b,
           router_w, router_b, gp_w, gp_b):
    B, T, C = data.shape
    d16 = data.astype(_BF)
    wq16 = in_proj_w[:C].astype(_BF)
    wk16 = in_proj_w[C:2 * C].astype(_BF)
    wv16 = in_proj_w[2 * C:].astype(_BF)
    outw16 = out_w.astype(_BF)
    rw16 = router_w.astype(_BF)
    gcw16 = gp_w[0::2].astype(_BF)
    gww16 = gp_w[1::2].astype(_BF)
    qst16 = qst.astype(_BF)
    bq = in_proj_b[:C].reshape(1, C)
    bk = in_proj_b[C:2 * C].reshape(1, C)
    bv = in_proj_b[2 * C:].reshape(1, C)
    outb = out_b.reshape(1, C)
    rb = router_b.reshape(1, _E)
    gcb = gp_b[0::2].reshape(1, _E)
    gwb = gp_b[1::2].reshape(1, _E)
    base = jnp.linspace(_MARGIN, 1.0 - _MARGIN, _E).astype(_F32).reshape(1, _E)

    q = pl.pallas_call(
        _q_body,
        out_shape=jax.ShapeDtypeStruct((B, C), _F32),
    )(qst16, wq16, bq)
    qh16 = q.astype(_BF).reshape(B, 1, C)

    ctx = pl.pallas_call(
        _attn_body,
        grid=(B,),
        in_specs=[
            pl.BlockSpec((1, T, C), lambda b: (b, 0, 0)),
            pl.BlockSpec((C, C), lambda b: (0, 0)),
            pl.BlockSpec((C, C), lambda b: (0, 0)),
            pl.BlockSpec((1, 1, C), lambda b: (b, 0, 0)),
            pl.BlockSpec((1, C), lambda b: (0, 0)),
            pl.BlockSpec((1, C), lambda b: (0, 0)),
        ],
        out_specs=pl.BlockSpec((1, 1, C), lambda b: (b, 0, 0)),
        out_shape=jax.ShapeDtypeStruct((B, 1, C), _F32),
        scratch_shapes=[pltpu.VMEM((T, _H), _F32)],
    )(d16, wk16, wv16, qh16, bk, bv)
    ctx16 = ctx.reshape(B, C).astype(_BF)

    logits, c, iw, pk = pl.pallas_call(
        functools.partial(_head_body, float(T)),
        out_shape=[
            jax.ShapeDtypeStruct((B, _E), _F32),
            jax.ShapeDtypeStruct((B, _E), _F32),
            jax.ShapeDtypeStruct((B, _E), _F32),
            jax.ShapeDtypeStruct((B, _E), _F32),
        ],
        scratch_shapes=[pltpu.VMEM((B, C), _F32)],
    )(ctx16, outw16, outb, rw16, rb, gcw16, gcb, gww16, gwb, base)

    weight = _make_sc_weight(B, T)(logits, c, iw, pk)
    return weight
